# baseline (speedup 1.0000x reference)
import functools

import jax
import jax.numpy as jnp
from jax.experimental import pallas as pl
from jax.experimental.pallas import tpu as pltpu

_BM = 1024
_BH = 512


def _dot_dims():
    return (((0,), (0,)), ((), ()))


def _tree_colmin(dg):
    n = dg.shape[0]
    while n > 256:
        n //= 4
        dg = jnp.minimum(jnp.minimum(dg[:n], dg[n:2 * n]),
                         jnp.minimum(dg[2 * n:3 * n], dg[3 * n:]))
    return jnp.min(dg, axis=0, keepdims=True)


def _reduce_half(d):
    folds = []
    csum = None
    for g in range(_BH // 128):
        dg = d[:, g * 128:(g + 1) * 128]
        cm = jnp.maximum(_tree_colmin(dg), 0.0)
        csum = cm if csum is None else csum + cm
        folds.append(dg)
    while len(folds) > 1:
        folds = [jnp.minimum(folds[i], folds[i + 1])
                 for i in range(0, len(folds), 2)]
    return folds[0], csum


def _chamfer_body(xpa_ref, ypa_ref, rowsum_ref, colsum_ref,
                  z0a_ref, z1a_ref, z0b_ref, z1b_ref,
                  rowacca_ref, rowaccb_ref, csuma_ref, csumb_ref,
                  *, num_mblocks):
    j = pl.program_id(1)
    xa, ya = xpa_ref[0], ypa_ref[0]
    xb, yb = xpa_ref[1], ypa_ref[1]

    f1a, s1a = _reduce_half(z1a_ref[...])
    f1b, s1b = _reduce_half(z1b_ref[...])

    z0a_ref[...] = jax.lax.dot_general(xa, ya[:, :_BH], _dot_dims(),
                                       preferred_element_type=jnp.float32)
    z0b_ref[...] = jax.lax.dot_general(xb, yb[:, :_BH], _dot_dims(),
                                       preferred_element_type=jnp.float32)
    z1a_ref[...] = jax.lax.dot_general(xa, ya[:, _BH:], _dot_dims(),
                                       preferred_element_type=jnp.float32)
    z1b_ref[...] = jax.lax.dot_general(xb, yb[:, _BH:], _dot_dims(),
                                       preferred_element_type=jnp.float32)

    f0a, s0a = _reduce_half(z0a_ref[...])
    f0b, s0b = _reduce_half(z0b_ref[...])

    acca = jnp.minimum(rowacca_ref[...], jnp.minimum(f0a, f1a))
    rowacca_ref[...] = jnp.where(j == 0, f0a, acca)
    csuma_ref[...] = jnp.where(j == 0, s0a, csuma_ref[...] + s0a + s1a)
    accb = jnp.minimum(rowaccb_ref[...], jnp.minimum(f0b, f1b))
    rowaccb_ref[...] = jnp.where(j == 0, f0b, accb)
    csumb_ref[...] = jnp.where(j == 0, s0b, csumb_ref[...] + s0b + s1b)

    @pl.when(j == num_mblocks - 1)
    def _finish():
        for p, z1_ref, rowacc_ref, csum_ref in (
                (0, z1a_ref, rowacca_ref, csuma_ref),
                (1, z1b_ref, rowaccb_ref, csumb_ref)):
            f1c, s1c = _reduce_half(z1_ref[...])
            rowall = jnp.minimum(rowacc_ref[...], f1c)
            rowmin = jnp.min(rowall, axis=1, keepdims=True)
            rowclamped = jnp.maximum(rowmin, 0.0)
            rowsum_ref[p, :, :] = jnp.sum(rowclamped, axis=0, keepdims=True)
            colsum_ref[p, :, :] = jnp.sum(csum_ref[...] + s1c, axis=1,
                                          keepdims=True)


def _hi_lo(v):
    hi = jax.lax.bitcast_convert_type(
        jax.lax.bitcast_convert_type(v, jnp.uint32) & jnp.uint32(0xFFFF0000),
        jnp.float32)
    return hi, v - hi


def kernel(x, y):
    B, N, D = x.shape
    M = y.shape[1]
    num_mblocks = M // _BM

    x2 = jnp.sum(x * x, axis=-1)[:, None, :]
    y2 = jnp.sum(y * y, axis=-1)[:, None, :]
    x2hi, x2lo = _hi_lo(x2)
    y2hi, y2lo = _hi_lo(y2)
    ones_n = jnp.ones((B, 1, N), jnp.float32)
    ones_m = jnp.ones((B, 1, M), jnp.float32)
    xpa = jnp.concatenate(
        [jnp.swapaxes(-2.0 * x, 1, 2), x2hi, x2lo, ones_n, ones_n],
        axis=1).astype(jnp.bfloat16)
    ypa = jnp.concatenate(
        [jnp.swapaxes(y, 1, 2), ones_m, ones_m, y2hi, y2lo],
        axis=1).astype(jnp.bfloat16)

    nj = num_mblocks
    yidx = lambda b, j: (b, 0, j)

    rowsums, colsums = pl.pallas_call(
        functools.partial(_chamfer_body, num_mblocks=num_mblocks),
        grid=(B // 2, nj),
        in_specs=[
            pl.BlockSpec((2, 7, N), lambda b, j: (b, 0, 0)),
            pl.BlockSpec((2, 7, _BM), yidx),
        ],
        out_specs=[
            pl.BlockSpec((2, 1, 1), lambda b, j: (b, 0, 0)),
            pl.BlockSpec((2, 1, 1), lambda b, j: (b, 0, 0)),
        ],
        out_shape=[
            jax.ShapeDtypeStruct((B, 1, 1), jnp.float32),
            jax.ShapeDtypeStruct((B, 1, 1), jnp.float32),
        ],
        scratch_shapes=[
            pltpu.VMEM((N, _BH), jnp.float32),
            pltpu.VMEM((N, _BH), jnp.float32),
            pltpu.VMEM((N, _BH), jnp.float32),
            pltpu.VMEM((N, _BH), jnp.float32),
            pltpu.VMEM((N, 128), jnp.float32),
            pltpu.VMEM((N, 128), jnp.float32),
            pltpu.VMEM((1, 128), jnp.float32),
            pltpu.VMEM((1, 128), jnp.float32),
        ],
        compiler_params=pltpu.CompilerParams(
            dimension_semantics=("parallel", "arbitrary")),
    )(xpa, ypa)

    return (jnp.sum(rowsums) / (B * N)) + (jnp.sum(colsums) / (B * M))
